# baseline (speedup 1.0000x reference)
import functools

import jax
import jax.numpy as jnp
from jax import lax
from jax.experimental import pallas as pl
from jax.experimental.pallas import tpu as pltpu
from jax.experimental.pallas import tpu_sc as plsc

VOCAB = 1_000_000
D = 32
B = 4096
L = 200

NC = 2
NS = 16
NW = NC * NS
BW = B // NW

_mesh = plsc.VectorSubcoreMesh(core_axis_name="c", subcore_axis_name="s")


@functools.partial(
    pl.kernel,
    out_type=[
        jax.ShapeDtypeStruct((L, D, B), jnp.float32),
        jax.ShapeDtypeStruct((D, B), jnp.float32),
    ],
    mesh=_mesh,
    compiler_params=pltpu.CompilerParams(
        use_tc_tiling_on_sc=False, needs_layout_passes=False
    ),
    scratch_types=[
        pltpu.VMEM((L, BW), jnp.int32),
        pltpu.VMEM((BW, D), jnp.float32),
        pltpu.VMEM((BW, D), jnp.float32),
        pltpu.VMEM((D, BW), jnp.float32),
        pltpu.VMEM((D, BW), jnp.float32),
        pltpu.VMEM((D,), jnp.int32),
        pltpu.VMEM_SHARED((NS * D, BW), jnp.float32),
        pltpu.SemaphoreType.DMA,
        pltpu.SemaphoreType.DMA,
        pltpu.SemaphoreType.DMA,
        pltpu.SemaphoreType.DMA,
        pltpu.SemaphoreType.DMA,
        pltpu.SemaphoreType.DMA,
    ],
)
def _sc_embed(ids_hbm, table_hbm, seq_hbm, pool_hbm,
              idsv, rows0, rows1, col0, col1, idxp, accsh,
              sg0, sg1, sw0, sw1, sp0, sp1):
    sid = lax.axis_index("s")
    wid = sid * NC + lax.axis_index("c")
    b0 = wid * BW

    pltpu.sync_copy(ids_hbm.at[:, pl.ds(b0, BW)], idsv)

    base16 = lax.iota(jnp.int32, 16)
    idxp[pl.ds(0, 16)] = base16 + sid * D
    idxp[pl.ds(16, 16)] = base16 + (sid * D + 16)

    zero = jnp.zeros((16,), jnp.float32)

    def zero_body(i, _):
        col0[i // (BW // 16), pl.ds((i % (BW // 16)) * 16, 16)] = zero
        return 0

    lax.fori_loop(0, D * BW // 16, zero_body, 0)
    pltpu.sync_copy(col0, accsh.at[pl.ds(sid * D, D)])

    rowidx = [base16 + (g * 16) for g in range(BW // 16)]

    def fire_gather(t, rows_b, sem):
        pltpu.async_copy(table_hbm.at[idsv.at[t]], rows_b, sem)

    def drain_gather(rows_b, sem):
        pltpu.make_async_copy(table_hbm.at[idsv.at[0]], rows_b, sem).wait()

    def fire_out(t, col_b, sem):
        pltpu.async_copy(col_b, seq_hbm.at[t, :, pl.ds(b0, BW)], sem)

    def drain_out(col_b, sem):
        pltpu.make_async_copy(col_b, seq_hbm.at[0, :, pl.ds(b0, BW)], sem).wait()

    def fire_pool(col_b, sem):
        pltpu.async_copy(col_b, accsh.at[idxp], sem, add=True)

    def drain_pool(col_b, sem):
        pltpu.make_async_copy(col_b, accsh.at[idxp], sem).wait()

    def transpose(rows_b, col_b):
        for f in range(D):
            cf = jnp.full((16,), f, jnp.int32)
            for g in range(0, BW // 16, 2):
                va = plsc.load_gather(rows_b, [rowidx[g], cf])
                vb = plsc.load_gather(rows_b, [rowidx[g + 1], cf])
                col_b[f, pl.ds(g * 16, 16)] = va
                col_b[f, pl.ds(g * 16 + 16, 16)] = vb

    fire_gather(0, rows0, sg0)

    def pair_body(p, _):
        t0 = 2 * p
        @pl.when(p > 0)
        def _():
            drain_out(col0, sw0)
            drain_pool(col0, sp0)
        drain_gather(rows0, sg0)
        fire_gather(t0 + 1, rows1, sg1)
        transpose(rows0, col0)
        fire_out(t0, col0, sw0)
        fire_pool(col0, sp0)
        @pl.when(p > 0)
        def _():
            drain_out(col1, sw1)
            drain_pool(col1, sp1)
        drain_gather(rows1, sg1)

        @pl.when(p < L // 2 - 1)
        def _():
            fire_gather(t0 + 2, rows0, sg0)
        transpose(rows1, col1)
        fire_out(t0 + 1, col1, sw1)
        fire_pool(col1, sp1)
        return 0

    lax.fori_loop(0, L // 2, pair_body, 0)
    drain_out(col0, sw0)
    drain_pool(col0, sp0)
    drain_out(col1, sw1)
    drain_pool(col1, sp1)

    pltpu.sync_copy(accsh.at[pl.ds(sid * D, D)], col0)
    inv = 1.0 / L

    def scale_body(i, _):
        f = i // (BW // 16)
        o = (i % (BW // 16)) * 16
        col0[f, pl.ds(o, 16)] = col0[f, pl.ds(o, 16)] * inv
        return 0

    lax.fori_loop(0, D * BW // 16, scale_body, 0)
    pltpu.sync_copy(col0, pool_hbm.at[:, pl.ds(b0, BW)])


def kernel(input_ids, embeddings):
    ids_t = input_ids.T.astype(jnp.int32)
    seq_tfb, pooled_fb = _sc_embed(ids_t, embeddings)
    seq = jnp.transpose(seq_tfb, (2, 0, 1))
    pooled = pooled_fb.T
    return seq, pooled


# speedup vs baseline: 1.1116x; 1.1116x over previous
import functools

import jax
import jax.numpy as jnp
from jax import lax
from jax.experimental import pallas as pl
from jax.experimental.pallas import tpu as pltpu
from jax.experimental.pallas import tpu_sc as plsc

VOCAB = 1_000_000
D = 32
B = 4096
L = 200

NC = 2
NS = 16
NW = NC * NS
BW = B // NW

_mesh = plsc.VectorSubcoreMesh(core_axis_name="c", subcore_axis_name="s")


@functools.partial(
    pl.kernel,
    out_type=[
        jax.ShapeDtypeStruct((L, B, D), jnp.float32),
        jax.ShapeDtypeStruct((B, D), jnp.float32),
    ],
    mesh=_mesh,
    compiler_params=pltpu.CompilerParams(
        use_tc_tiling_on_sc=False, needs_layout_passes=False
    ),
    scratch_types=[
        pltpu.VMEM((L, BW), jnp.int32),
        pltpu.VMEM((BW, D), jnp.float32),
        pltpu.VMEM((BW, D), jnp.float32),
        pltpu.VMEM((BW,), jnp.int32),
        pltpu.VMEM_SHARED((NS * BW, D), jnp.float32),
        pltpu.SemaphoreType.DMA,
        pltpu.SemaphoreType.DMA,
        pltpu.SemaphoreType.DMA,
        pltpu.SemaphoreType.DMA,
        pltpu.SemaphoreType.DMA,
        pltpu.SemaphoreType.DMA,
    ],
)
def _sc_embed(ids_hbm, table_hbm, seq_hbm, pool_hbm,
              idsv, rows0, rows1, idxp, accsh,
              sg0, sg1, sw0, sw1, sp0, sp1):
    sid = lax.axis_index("s")
    wid = sid * NC + lax.axis_index("c")
    b0 = wid * BW

    pltpu.sync_copy(ids_hbm.at[:, pl.ds(b0, BW)], idsv)

    base16 = lax.iota(jnp.int32, 16)
    for g in range(BW // 16):
        idxp[pl.ds(g * 16, 16)] = base16 + (sid * BW + g * 16)

    zero = jnp.zeros((16,), jnp.float32)

    def zero_body(i, _):
        rows0[i // 2, pl.ds((i % 2) * 16, 16)] = zero
        return 0

    lax.fori_loop(0, BW * D // 16, zero_body, 0)
    pltpu.sync_copy(rows0, accsh.at[pl.ds(sid * BW, BW)])

    def fire_gather(t, rows_b, sem):
        pltpu.async_copy(table_hbm.at[idsv.at[t]], rows_b, sem)

    def drain_gather(rows_b, sem):
        pltpu.make_async_copy(table_hbm.at[idsv.at[0]], rows_b, sem).wait()

    def fire_out(t, rows_b, sem):
        pltpu.async_copy(rows_b, seq_hbm.at[t, pl.ds(b0, BW)], sem)

    def drain_out(rows_b, sem):
        pltpu.make_async_copy(rows_b, seq_hbm.at[0, pl.ds(b0, BW)], sem).wait()

    def fire_pool(rows_b, sem):
        pltpu.async_copy(rows_b, accsh.at[idxp], sem, add=True)

    def drain_pool(rows_b, sem):
        pltpu.make_async_copy(rows_b, accsh.at[idxp], sem).wait()

    fire_gather(0, rows0, sg0)

    def pair_body(p, _):
        t0 = 2 * p
        @pl.when(p > 0)
        def _():
            drain_out(rows0, sw0)
            drain_pool(rows0, sp0)
        drain_gather(rows0, sg0)
        fire_gather(t0 + 1, rows1, sg1)
        fire_out(t0, rows0, sw0)
        fire_pool(rows0, sp0)
        @pl.when(p > 0)
        def _():
            drain_out(rows1, sw1)
            drain_pool(rows1, sp1)
        drain_gather(rows1, sg1)

        @pl.when(p < L // 2 - 1)
        def _():
            fire_gather(t0 + 2, rows0, sg0)
        fire_out(t0 + 1, rows1, sw1)
        fire_pool(rows1, sp1)
        return 0

    lax.fori_loop(0, L // 2, pair_body, 0)
    drain_out(rows0, sw0)
    drain_pool(rows0, sp0)
    drain_out(rows1, sw1)
    drain_pool(rows1, sp1)

    pltpu.sync_copy(accsh.at[pl.ds(sid * BW, BW)], rows0)
    inv = 1.0 / L

    def scale_body(i, _):
        b = i // 2
        s = pl.ds((i % 2) * 16, 16)
        rows0[b, s] = rows0[b, s] * inv
        return 0

    lax.fori_loop(0, BW * D // 16, scale_body, 0)
    pltpu.sync_copy(rows0, pool_hbm.at[pl.ds(b0, BW)])


def kernel(input_ids, embeddings):
    ids_t = input_ids.T.astype(jnp.int32)
    seq_tbf, pooled = _sc_embed(ids_t, embeddings)
    seq = jnp.transpose(seq_tbf, (1, 0, 2))
    return seq, pooled


# speedup vs baseline: 1.1751x; 1.0571x over previous
import functools

import jax
import jax.numpy as jnp
from jax import lax
from jax.experimental import pallas as pl
from jax.experimental.pallas import tpu as pltpu
from jax.experimental.pallas import tpu_sc as plsc

VOCAB = 1_000_000
D = 32
B = 4096
L = 200

NC = 2
NS = 16
NW = NC * NS
SW = B // NW

CS = 8
CHUNK_IDS = CS * L
G = 80
NG = CHUNK_IDS // G
NCHUNK = SW // CS

_mesh = plsc.VectorSubcoreMesh(core_axis_name="c", subcore_axis_name="s")


@functools.partial(
    pl.kernel,
    out_type=[
        jax.ShapeDtypeStruct((B * L, D), jnp.float32),
        jax.ShapeDtypeStruct((B, D), jnp.float32),
    ],
    mesh=_mesh,
    compiler_params=pltpu.CompilerParams(use_tc_tiling_on_sc=False),
    scratch_types=[
        pltpu.VMEM((CHUNK_IDS,), jnp.int32),
        pltpu.VMEM((CHUNK_IDS,), jnp.int32),
        pltpu.VMEM((CHUNK_IDS, D), jnp.float32),
        pltpu.VMEM((CHUNK_IDS, D), jnp.float32),
        pltpu.VMEM((CS, D), jnp.float32),
        pltpu.SemaphoreType.DMA,
        pltpu.SemaphoreType.DMA,
        pltpu.SemaphoreType.DMA,
        pltpu.SemaphoreType.DMA,
    ],
)
def _sc_embed(ids_hbm, table_hbm, seq_hbm, pool_hbm,
              idx0, idx1, rows0, rows1, pool_v, sg0, sg1, sw0, sw1):
    wid = lax.axis_index("s") * NC + lax.axis_index("c")
    idx = (idx0, idx1)
    rows = (rows0, rows1)
    sg = (sg0, sg1)
    sw = (sw0, sw1)

    def i_base(c):
        return (wid * SW + c * CS) * L

    def load_and_fire(c, b):
        pltpu.sync_copy(ids_hbm.at[pl.ds(i_base(c), CHUNK_IDS)], idx[b])
        for j in range(NG):
            pltpu.async_copy(
                table_hbm.at[idx[b].at[pl.ds(j * G, G)]],
                rows[b].at[pl.ds(j * G, G)],
                sg[b],
            )

    def drain_gathers(c, b):
        for j in range(NG):
            pltpu.make_async_copy(
                table_hbm.at[idx[b].at[pl.ds(j * G, G)]],
                rows[b].at[pl.ds(j * G, G)],
                sg[b],
            ).wait()

    def start_writeout(c, b):
        pltpu.async_copy(rows[b], seq_hbm.at[pl.ds(i_base(c), CHUNK_IDS)], sw[b])

    def wait_writeout(c, b):
        pltpu.make_async_copy(
            rows[b], seq_hbm.at[pl.ds(i_base(c), CHUNK_IDS)], sw[b]
        ).wait()

    def compute_pool(c, b):
        rows_b = rows[b]

        def sample_body(s, _):
            rb = s * L

            def row_body(i, carry):
                a0, a1 = carry
                r0 = rb + i * 8
                for r in range(8):
                    a0 = a0 + rows_b[r0 + r, pl.ds(0, 16)]
                    a1 = a1 + rows_b[r0 + r, pl.ds(16, 16)]
                return (a0, a1)

            z = jnp.zeros((16,), jnp.float32)
            a0, a1 = lax.fori_loop(0, L // 8, row_body, (z, z))
            pool_v[s, pl.ds(0, 16)] = a0 * (1.0 / L)
            pool_v[s, pl.ds(16, 16)] = a1 * (1.0 / L)
            return 0

        lax.fori_loop(0, CS, sample_body, 0)
        pltpu.sync_copy(pool_v, pool_hbm.at[pl.ds(wid * SW + c * CS, CS)])

    load_and_fire(0, 0)
    for c in range(NCHUNK):
        b = c & 1
        nb = 1 - b
        if c + 1 < NCHUNK:
            if c >= 1:
                wait_writeout(c - 1, nb)
            load_and_fire(c + 1, nb)
        drain_gathers(c, b)
        compute_pool(c, b)
        start_writeout(c, b)
    wait_writeout(NCHUNK - 2, (NCHUNK - 2) & 1)
    wait_writeout(NCHUNK - 1, (NCHUNK - 1) & 1)


def kernel(input_ids, embeddings):
    ids_flat = input_ids.reshape(-1).astype(jnp.int32)
    seq_flat, pooled = _sc_embed(ids_flat, embeddings)
    return seq_flat.reshape(B, L, D), pooled
